# baseline (speedup 1.0000x reference)
import functools

import jax
import jax.numpy as jnp
from jax import lax
from jax.experimental import pallas as pl
from jax.experimental.pallas import tpu as pltpu
from jax.experimental.pallas import tpu_sc as plsc

D0, D1, D2 = 100, 100, 100
B = 16384
NC, NS, L = 2, 16, 16
NW = NC * NS
BPW = B // NW
GROUPS = BPW // L

_mesh = plsc.VectorSubcoreMesh(core_axis_name="c", subcore_axis_name="s")


def _worker_base():
    return (lax.axis_index("s") * NC + lax.axis_index("c")) * BPW


@functools.partial(
    pl.kernel,
    mesh=_mesh,
    out_type=jax.ShapeDtypeStruct((B,), jnp.int32),
    scratch_types=[
        pltpu.VMEM((BPW,), jnp.int32),
        pltpu.VMEM((BPW,), jnp.int32),
        pltpu.VMEM((BPW,), jnp.int32),
        pltpu.VMEM((BPW,), jnp.int32),
        pltpu.SemaphoreType.DMA,
    ],
)
def _flatten(xt_hbm, idx_hbm, x0_v, x1_v, x2_v, idx_v, sem):
    base = _worker_base()
    cp0 = pltpu.async_copy(xt_hbm.at[pl.ds(0 * B + base, BPW)], x0_v, sem)
    cp1 = pltpu.async_copy(xt_hbm.at[pl.ds(1 * B + base, BPW)], x1_v, sem)
    cp2 = pltpu.async_copy(xt_hbm.at[pl.ds(2 * B + base, BPW)], x2_v, sem)
    cp0.wait()
    cp1.wait()
    cp2.wait()

    def group(g, carry):
        s = pl.ds(g * L, L)
        idx_v[s] = x0_v[s] + x1_v[s] * D0 + x2_v[s] * (D0 * D1)
        return carry

    lax.fori_loop(0, GROUPS, group, 0)
    pltpu.sync_copy(idx_v, idx_hbm.at[pl.ds(base, BPW)])


@functools.partial(
    pl.kernel,
    mesh=_mesh,
    out_type=jax.ShapeDtypeStruct((B,), jnp.float32),
    scratch_types=[
        pltpu.VMEM((BPW,), jnp.int32),
        pltpu.VMEM((BPW,), jnp.float32),
        pltpu.SemaphoreType.DMA,
    ],
)
def _gather(w_hbm, idx_hbm, out_hbm, idx_v, val_v, sem):
    base = _worker_base()
    pltpu.sync_copy(idx_hbm.at[pl.ds(base, BPW)], idx_v)
    pltpu.async_copy(w_hbm.at[idx_v], val_v, sem).wait()
    pltpu.sync_copy(val_v, out_hbm.at[pl.ds(base, BPW)])


def kernel(x, W):
    xt = x.astype(jnp.int32).T.reshape(-1)
    idx = _flatten(xt)
    wf = W.transpose(2, 1, 0).reshape(-1)
    return _gather(wf, idx).reshape(B, 1)


# speedup vs baseline: 1.1347x; 1.1347x over previous
import functools

import jax
import jax.numpy as jnp
from jax import lax
from jax.experimental import pallas as pl
from jax.experimental.pallas import tpu as pltpu
from jax.experimental.pallas import tpu_sc as plsc

D0, D1, D2 = 100, 100, 100
B = 16384
NC, NS, L = 2, 16, 16
NW = NC * NS
BPW = B // NW
GROUPS = BPW // L

_mesh = plsc.VectorSubcoreMesh(core_axis_name="c", subcore_axis_name="s")


@functools.partial(
    pl.kernel,
    mesh=_mesh,
    out_type=jax.ShapeDtypeStruct((B,), jnp.float32),
    scratch_types=[
        pltpu.VMEM((BPW,), jnp.int32),
        pltpu.VMEM((BPW,), jnp.int32),
        pltpu.VMEM((BPW,), jnp.int32),
        pltpu.VMEM((BPW,), jnp.int32),
        pltpu.VMEM((BPW,), jnp.float32),
        pltpu.SemaphoreType.DMA,
    ],
)
def _clique_gather(xt_hbm, w_hbm, out_hbm, x0_v, x1_v, x2_v, idx_v, val_v,
                   sem):
    wid = lax.axis_index("s") * NC + lax.axis_index("c")
    base = wid * BPW
    cp0 = pltpu.async_copy(xt_hbm.at[pl.ds(0 * B + base, BPW)], x0_v, sem)
    cp1 = pltpu.async_copy(xt_hbm.at[pl.ds(1 * B + base, BPW)], x1_v, sem)
    cp2 = pltpu.async_copy(xt_hbm.at[pl.ds(2 * B + base, BPW)], x2_v, sem)
    cp0.wait()
    cp1.wait()
    cp2.wait()

    def group(g, carry):
        s = pl.ds(g * L, L)
        idx_v[s] = x0_v[s] * (D1 * D2) + x1_v[s] * D2 + x2_v[s]
        return carry

    lax.fori_loop(0, GROUPS, group, 0)
    pltpu.async_copy(w_hbm.at[idx_v], val_v, sem).wait()
    pltpu.sync_copy(val_v, out_hbm.at[pl.ds(base, BPW)])


def kernel(x, W):
    xt = x.astype(jnp.int32).T.reshape(-1)
    wf = W.reshape(-1)
    return _clique_gather(xt, wf).reshape(B, 1)
